# baseline (speedup 1.0000x reference)
import functools

import jax
import jax.numpy as jnp
from jax import lax
from jax.experimental import pallas as pl
from jax.experimental.pallas import tpu as pltpu

_PACK = 8
_ROW_PAD = 512


def _gemp_body(x_ref, w_ref, o_ref, *, p, eps, inv_n, row_valid):
    x = x_ref[...]
    lane = lax.broadcasted_iota(jnp.int32, x.shape, 1)
    x = jnp.where(lane < row_valid, x, 0.0)
    x3 = (x * x * x).astype(jnp.bfloat16)
    s = lax.dot_general(
        x3, w_ref[...], (((1,), (0,)), ((), ())),
        preferred_element_type=jnp.float32)
    o_ref[...] = ((s * inv_n + eps) ** (1.0 / p)).astype(o_ref.dtype)


def kernel(x):
    p, eps = 3.0, 1e-12
    B, C, H, W = x.shape
    n = H * W
    R = B * C
    rows = R // _PACK
    row_valid = _PACK * n
    xr = x.reshape(rows, row_valid)

    j = jnp.arange(_ROW_PAD)[:, None]
    seg = jnp.arange(_PACK)[None, :]
    wt = ((j >= seg * n) & (j < (seg + 1) * n)).astype(jnp.bfloat16)

    tr = min(1024, rows)
    body = functools.partial(_gemp_body, p=p, eps=eps, inv_n=1.0 / n,
                             row_valid=row_valid)
    out = pl.pallas_call(
        body,
        out_shape=jax.ShapeDtypeStruct((rows, _PACK), x.dtype),
        grid=(rows // tr,),
        in_specs=[
            pl.BlockSpec((tr, _ROW_PAD), lambda r: (r, 0)),
            pl.BlockSpec((_ROW_PAD, _PACK), lambda r: (0, 0)),
        ],
        out_specs=pl.BlockSpec((tr, _PACK), lambda r: (r, 0)),
        compiler_params=pltpu.CompilerParams(
            dimension_semantics=("parallel",)),
    )(xr, wt)
    return out.reshape(B, C)


# speedup vs baseline: 40.7382x; 40.7382x over previous
import functools

import jax
import jax.numpy as jnp
from jax.experimental import pallas as pl
from jax.experimental.pallas import tpu as pltpu

_TB = 8


def _gemp_body(y_ref, o_ref, *, p, eps, inv_n):
    v = y_ref[...]
    s = jnp.sum(v * v * v, axis=0)
    o_ref[...] = ((s * inv_n + eps) ** (1.0 / p)).astype(o_ref.dtype)


def kernel(x):
    p, eps = 3.0, 1e-12
    B, C, H, W = x.shape
    n = H * W
    y = x.transpose(2, 3, 0, 1).reshape(n, B, C)

    body = functools.partial(_gemp_body, p=p, eps=eps, inv_n=1.0 / n)
    return pl.pallas_call(
        body,
        out_shape=jax.ShapeDtypeStruct((B, C), x.dtype),
        grid=(B // _TB,),
        in_specs=[pl.BlockSpec((n, _TB, C), lambda b: (0, b, 0))],
        out_specs=pl.BlockSpec((_TB, C), lambda b: (b, 0)),
        compiler_params=pltpu.CompilerParams(
            dimension_semantics=("parallel",)),
    )(y)


# speedup vs baseline: 49.0988x; 1.2052x over previous
import functools

import jax
import jax.numpy as jnp
from jax.experimental import pallas as pl
from jax.experimental.pallas import tpu as pltpu

_TB = 16


def _gemp_body(y_ref, o_ref, *, p, eps, inv_n):
    v = y_ref[...]
    s = jnp.sum(v * v * v, axis=0)
    o_ref[...] = ((s * inv_n + eps) ** (1.0 / p)).astype(o_ref.dtype)


def kernel(x):
    p, eps = 3.0, 1e-12
    B, C, H, W = x.shape
    n = H * W
    y = x.transpose(2, 3, 0, 1).reshape(n, B, C)

    body = functools.partial(_gemp_body, p=p, eps=eps, inv_n=1.0 / n)
    return pl.pallas_call(
        body,
        out_shape=jax.ShapeDtypeStruct((B, C), x.dtype),
        grid=(B // _TB,),
        in_specs=[pl.BlockSpec((n, _TB, C), lambda b: (0, b, 0))],
        out_specs=pl.BlockSpec((_TB, C), lambda b: (b, 0)),
        compiler_params=pltpu.CompilerParams(
            dimension_semantics=("parallel",)),
    )(y)


# speedup vs baseline: 49.6539x; 1.0113x over previous
import functools

import jax
import jax.numpy as jnp
from jax.experimental import pallas as pl
from jax.experimental.pallas import tpu as pltpu

_TB = 32


def _gemp_body(y_ref, o_ref, *, p, eps, inv_n):
    v = y_ref[...]
    s = jnp.sum(v * v * v, axis=0)
    o_ref[...] = ((s * inv_n + eps) ** (1.0 / p)).astype(o_ref.dtype)


def kernel(x):
    p, eps = 3.0, 1e-12
    B, C, H, W = x.shape
    n = H * W
    y = x.transpose(2, 3, 0, 1).reshape(n, B, C)

    body = functools.partial(_gemp_body, p=p, eps=eps, inv_n=1.0 / n)
    return pl.pallas_call(
        body,
        out_shape=jax.ShapeDtypeStruct((B, C), x.dtype),
        grid=(B // _TB,),
        in_specs=[pl.BlockSpec((n, _TB, C), lambda b: (0, b, 0))],
        out_specs=pl.BlockSpec((_TB, C), lambda b: (b, 0)),
        compiler_params=pltpu.CompilerParams(
            dimension_semantics=("parallel",)),
    )(y)
